# baseline (speedup 1.0000x reference)
import functools

import jax
import jax.numpy as jnp
from jax import lax
from jax.experimental import pallas as pl
from jax.experimental.pallas import tpu as pltpu
from jax.experimental.pallas import tpu_sc as plsc

E = 64
TOPK = 2
NG = 8
GSZ = E // NG
TKG = 4
D = 1024
FF = 256
CAP = 256
B = 1
S = 2048
T = B * S
P = T * TOPK

TB = 1024
NTB = T // TB

TILE = 64
TPE = CAP // TILE
NT = (E * CAP) // TILE
TRASH = E * CAP
XROWS = E * CAP + TILE

NWORK = 32
PPW = P // NWORK
TPW = T // NWORK
CH = 16


def _first_argmax(vals, iota, width):
    m = jnp.max(vals, axis=1, keepdims=True)
    eq = vals == m
    fidx = jnp.min(jnp.where(eq, iota, width), axis=1, keepdims=True)
    sel = iota == fidx
    return fidx, sel


def _route_body(x_ref, gw_ref, b_ref, sg_ref, su_ref, sd_ref,
                lo_ref, tk_ref, de_ref, wa_ref, wb_ref, cnt_ref, sh_ref,
                run_ref):
    step = pl.program_id(0)

    @pl.when(step == 0)
    def _():
        run_ref[...] = jnp.zeros_like(run_ref)

    x = x_ref[...]
    logits = lax.dot_general(x, gw_ref[...], (((1,), (1,)), ((), ())),
                             preferred_element_type=jnp.float32)
    lo_ref[...] = logits
    scores = 1.0 / (1.0 + jnp.exp(-logits))
    s_rt = scores + b_ref[...]

    iota8 = lax.broadcasted_iota(jnp.int32, (TB, GSZ), 1)
    gcols = []
    for g in range(NG):
        blk = s_rt[:, g * GSZ:(g + 1) * GSZ]
        m1 = jnp.max(blk, axis=1, keepdims=True)
        _, sel = _first_argmax(blk, iota8, GSZ)
        m2 = jnp.max(jnp.where(sel, -jnp.inf, blk), axis=1, keepdims=True)
        gcols.append(m1 + m2)
    gs = jnp.concatenate(gcols, axis=1)

    iota_g = lax.broadcasted_iota(jnp.int32, (TB, NG), 1)
    gmask = jnp.zeros((TB, NG), jnp.float32)
    for _ in range(TKG):
        _, sel = _first_argmax(gs, iota_g, NG)
        gmask = jnp.where(sel, 1.0, gmask)
        gs = jnp.where(sel, -jnp.inf, gs)

    cols = []
    for g in range(NG):
        keep = gmask[:, g:g + 1] > 0
        cols.append(jnp.where(keep, s_rt[:, g * GSZ:(g + 1) * GSZ], -jnp.inf))
    masked = jnp.concatenate(cols, axis=1)

    iota_e = lax.broadcasted_iota(jnp.int32, (TB, E), 1)
    idxs, scs = [], []
    for _ in range(TOPK):
        fidx, sel = _first_argmax(masked, iota_e, E)
        idxs.append(fidx)
        scs.append(jnp.sum(jnp.where(sel, scores, 0.0), axis=1, keepdims=True))
        masked = jnp.where(sel, -jnp.inf, masked)
    topk = jnp.concatenate(idxs, axis=1)
    tk_ref[...] = topk
    sc = jnp.concatenate(scs, axis=1)
    tw = sc / (jnp.sum(sc, axis=1, keepdims=True) + 1e-20)

    e0, e1 = idxs[0], idxs[1]
    oh0 = (iota_e == e0).astype(jnp.float32)
    oh1 = (iota_e == e1).astype(jnp.float32)
    ohsum = oh0 + oh1
    r_iota = lax.broadcasted_iota(jnp.int32, (TB, TB), 0)
    c_iota = lax.broadcasted_iota(jnp.int32, (TB, TB), 1)
    tril = (r_iota > c_iota).astype(jnp.float32)
    prev = jnp.dot(tril, ohsum, preferred_element_type=jnp.float32)
    prev = prev + run_ref[...]
    rank0 = jnp.sum(prev * oh0, axis=1, keepdims=True).astype(jnp.int32)
    rank1 = jnp.sum(prev * oh1, axis=1, keepdims=True).astype(jnp.int32)
    run_ref[...] = run_ref[...] + jnp.sum(ohsum, axis=0, keepdims=True)

    v0 = rank0 < CAP
    v1 = rank1 < CAP
    dest0 = jnp.where(v0, e0 * CAP + rank0, TRASH)
    dest1 = jnp.where(v1, e1 * CAP + rank1, TRASH)
    de_ref[...] = jnp.concatenate([dest0, dest1], axis=1)
    w0 = tw[:, 0:1] * v0.astype(jnp.float32)
    w1 = tw[:, 1:2] * v1.astype(jnp.float32)
    wa_ref[...] = jnp.broadcast_to(w0, (TB, 16))
    wb_ref[...] = jnp.broadcast_to(w1, (TB, 16))
    cnt_ref[...] = jnp.broadcast_to(run_ref[...], cnt_ref.shape).astype(jnp.int32)

    g_sh = jnp.dot(x, sg_ref[...], preferred_element_type=jnp.float32)
    u_sh = jnp.dot(x, su_ref[...], preferred_element_type=jnp.float32)
    h_sh = (g_sh * (1.0 / (1.0 + jnp.exp(-g_sh)))) * u_sh
    sh_ref[...] = jnp.dot(h_sh, sd_ref[...], preferred_element_type=jnp.float32)


def _route_call(x2d, gw, bias, Sg, Su, Sd):
    return pl.pallas_call(
        _route_body,
        grid=(NTB,),
        in_specs=[
            pl.BlockSpec((TB, D), lambda i: (i, 0)),
            pl.BlockSpec((E, D), lambda i: (0, 0)),
            pl.BlockSpec((1, E), lambda i: (0, 0)),
            pl.BlockSpec((D, FF), lambda i: (0, 0)),
            pl.BlockSpec((D, FF), lambda i: (0, 0)),
            pl.BlockSpec((FF, D), lambda i: (0, 0)),
        ],
        out_specs=[
            pl.BlockSpec((TB, E), lambda i: (i, 0)),
            pl.BlockSpec((TB, TOPK), lambda i: (i, 0)),
            pl.BlockSpec((TB, TOPK), lambda i: (i, 0)),
            pl.BlockSpec((TB, 16), lambda i: (i, 0)),
            pl.BlockSpec((TB, 16), lambda i: (i, 0)),
            pl.BlockSpec((8, E), lambda i: (0, 0)),
            pl.BlockSpec((TB, D), lambda i: (i, 0)),
        ],
        out_shape=[
            jax.ShapeDtypeStruct((T, E), jnp.float32),
            jax.ShapeDtypeStruct((T, TOPK), jnp.int32),
            jax.ShapeDtypeStruct((T, TOPK), jnp.int32),
            jax.ShapeDtypeStruct((T, 16), jnp.float32),
            jax.ShapeDtypeStruct((T, 16), jnp.float32),
            jax.ShapeDtypeStruct((8, E), jnp.int32),
            jax.ShapeDtypeStruct((T, D), jnp.float32),
        ],
        scratch_shapes=[pltpu.VMEM((1, E), jnp.float32)],
    )(x2d, gw, bias, Sg, Su, Sd)


def _dispatch_body(x_hbm, dest_hbm, xbuf_hbm,
                   dvm0, dvm1, rows0, rows1, sg0, sg1, ss0, ss1):
    wid = lax.axis_index("s") * 2 + lax.axis_index("c")
    lane = lax.iota(jnp.int32, 16)
    nch = PPW // CH
    dvms = [dvm0, dvm1]
    rows = [rows0, rows1]
    gsem = [sg0, sg1]
    ssem = [ss0, ss1]

    def issue_gather(c):
        base = wid * PPW + c * CH
        tok = (base + lane) & (T - 1)
        pltpu.sync_copy(dest_hbm.at[pl.ds(base, CH)], dvms[c % 2])
        return pltpu.async_copy(x_hbm.at[tok], rows[c % 2], gsem[c % 2])

    gh = [issue_gather(0), None]
    sh_ = [None, None]
    for c in range(nch):
        b = c % 2
        nb = (c + 1) % 2
        if c + 1 < nch:
            if sh_[nb] is not None:
                sh_[nb].wait()
            gh[nb] = issue_gather(c + 1)
        gh[b].wait()
        sh_[b] = pltpu.async_copy(rows[b], xbuf_hbm.at[dvms[b][...]], ssem[b])
    sh_[(nch - 1) % 2].wait()
    if sh_[nch % 2] is not None:
        sh_[nch % 2].wait()


def _dispatch_call(x2d, dest):
    mesh = plsc.VectorSubcoreMesh(core_axis_name="c", subcore_axis_name="s")
    f = functools.partial(
        pl.kernel,
        out_type=jax.ShapeDtypeStruct((XROWS, D), jnp.float32),
        mesh=mesh,
        scratch_types=[
            pltpu.VMEM((CH,), jnp.int32),
            pltpu.VMEM((CH,), jnp.int32),
            pltpu.VMEM((CH, D), jnp.float32),
            pltpu.VMEM((CH, D), jnp.float32),
            pltpu.SemaphoreType.DMA,
            pltpu.SemaphoreType.DMA,
            pltpu.SemaphoreType.DMA,
            pltpu.SemaphoreType.DMA,
        ],
    )(_dispatch_body)
    return f(x2d, dest)


def _ffn_body(cnt_ref, x_ref, wg_ref, wu_ref, wd_ref, o_ref):
    g = pl.program_id(0)
    e = jnp.minimum(g // TPE, E - 1)
    t = g % TPE
    is_trash = g >= NT
    active = jnp.logical_and(g < NT, cnt_ref[e] > t * TILE)

    @pl.when(is_trash)
    def _():
        o_ref[...] = jnp.zeros_like(o_ref)

    @pl.when(active)
    def _():
        x = x_ref[...]
        gg = jnp.dot(x, wg_ref[0], preferred_element_type=jnp.float32)
        uu = jnp.dot(x, wu_ref[0], preferred_element_type=jnp.float32)
        h = (gg * (1.0 / (1.0 + jnp.exp(-gg)))) * uu
        o_ref[...] = jnp.dot(h, wd_ref[0], preferred_element_type=jnp.float32)


def _ffn_xmap(g, cnt):
    e = jnp.minimum(g // TPE, E - 1)
    t = g % TPE
    last_act = jnp.maximum((cnt[e] + TILE - 1) // TILE - 1, 0)
    b = jnp.where(g >= NT, NT, TPE * e + jnp.minimum(t, last_act))
    return (b, 0)


def _ffn_call(counts, xbuf, Wg, Wu, Wd):
    grid_spec = pltpu.PrefetchScalarGridSpec(
        num_scalar_prefetch=1,
        grid=(NT + 1,),
        in_specs=[
            pl.BlockSpec((TILE, D), _ffn_xmap),
            pl.BlockSpec((1, D, FF), lambda g, c: (jnp.minimum(g // TPE, E - 1), 0, 0)),
            pl.BlockSpec((1, D, FF), lambda g, c: (jnp.minimum(g // TPE, E - 1), 0, 0)),
            pl.BlockSpec((1, FF, D), lambda g, c: (jnp.minimum(g // TPE, E - 1), 0, 0)),
        ],
        out_specs=pl.BlockSpec((TILE, D), _ffn_xmap),
    )
    return pl.pallas_call(
        _ffn_body,
        grid_spec=grid_spec,
        out_shape=jax.ShapeDtypeStruct((XROWS, D), jnp.float32),
    )(counts, xbuf, Wg, Wu, Wd)


def _combine_body(osort_hbm, dest_hbm, wa_hbm, wb_hbm, sh_hbm, y_hbm,
                  dva0, dva1, dvb0, dvb1, wva0, wva1, wvb0, wvb1,
                  ra0, ra1, rb0, rb1, rs0, rs1,
                  sa0, sa1, sb0, sb1, ss0, ss1, so0, so1):
    wid = lax.axis_index("s") * 2 + lax.axis_index("c")
    nch = TPW // CH
    dva = [dva0, dva1]
    dvb = [dvb0, dvb1]
    wva = [wva0, wva1]
    wvb = [wvb0, wvb1]
    ra = [ra0, ra1]
    rb = [rb0, rb1]
    rs = [rs0, rs1]
    sa = [sa0, sa1]
    sb = [sb0, sb1]
    ss = [ss0, ss1]
    so = [so0, so1]

    def issue(c):
        t0 = wid * TPW + c * CH
        b = c % 2
        pltpu.sync_copy(dest_hbm.at[pl.ds(t0, CH)], dva[b])
        pltpu.sync_copy(dest_hbm.at[pl.ds(T + t0, CH)], dvb[b])
        pltpu.sync_copy(wa_hbm.at[pl.ds(t0, CH)], wva[b])
        pltpu.sync_copy(wb_hbm.at[pl.ds(t0, CH)], wvb[b])
        return (pltpu.async_copy(osort_hbm.at[dva[b][...]], ra[b], sa[b]),
                pltpu.async_copy(osort_hbm.at[dvb[b][...]], rb[b], sb[b]),
                pltpu.async_copy(sh_hbm.at[pl.ds(t0, CH)], rs[b], ss[b]))

    hnd = [issue(0), None]
    out_h = [None, None]
    for c in range(nch):
        b = c % 2
        nb = (c + 1) % 2
        if c + 1 < nch:
            if out_h[nb] is not None:
                out_h[nb].wait()
            hnd[nb] = issue(c + 1)
        for h in hnd[b]:
            h.wait()
        for r in range(CH):
            wa = wva[b][r]
            wb = wvb[b][r]

            def body(j, _):
                sl = pl.ds(j * 16, 16)
                ra[b][r, sl] = wa * ra[b][r, sl] + wb * rb[b][r, sl] + rs[b][r, sl]
                return 0
            lax.fori_loop(0, D // 16, body, 0)
        t0 = wid * TPW + c * CH
        out_h[b] = pltpu.async_copy(ra[b], y_hbm.at[pl.ds(t0, CH)], so[b])
    out_h[(nch - 1) % 2].wait()
    if out_h[nch % 2] is not None:
        out_h[nch % 2].wait()


def _combine_call(osort, dest, wba, wbb, sh):
    mesh = plsc.VectorSubcoreMesh(core_axis_name="c", subcore_axis_name="s")
    f = functools.partial(
        pl.kernel,
        out_type=jax.ShapeDtypeStruct((T, D), jnp.float32),
        mesh=mesh,
        scratch_types=(
            [pltpu.VMEM((CH,), jnp.int32)] * 4
            + [pltpu.VMEM((CH, 16), jnp.float32)] * 4
            + [pltpu.VMEM((CH, D), jnp.float32)] * 6
            + [pltpu.SemaphoreType.DMA] * 8
        ),
    )(_combine_body)
    return f(osort, dest, wba, wbb, sh)


def kernel(hidden_states, gate_weight, expert_bias, Wg, Wu, Wd, Sg, Su, Sd):
    x2d = hidden_states.reshape(T, D)
    logits, topk, dest2d, wba, wbb, counts8, sh = _route_call(
        x2d, gate_weight, expert_bias.reshape(1, E), Sg, Su, Sd)
    dest = jnp.transpose(dest2d).reshape(P)
    counts = counts8[0]
    xbuf = _dispatch_call(x2d, dest)
    osort = _ffn_call(counts, xbuf, Wg, Wu, Wd)
    y = _combine_call(osort, dest, wba, wbb, sh)
    return (y.reshape(B, S, D), logits.reshape(B, S, E),
            topk.reshape(B, S, TOPK))


# speedup vs baseline: 1.0683x; 1.0683x over previous
import functools

import jax
import jax.numpy as jnp
from jax import lax
from jax.experimental import pallas as pl
from jax.experimental.pallas import tpu as pltpu
from jax.experimental.pallas import tpu_sc as plsc

E = 64
TOPK = 2
NG = 8
GSZ = E // NG
TKG = 4
D = 1024
FF = 256
CAP = 256
B = 1
S = 2048
T = B * S
P = T * TOPK

TB = 1024
NTB = T // TB

TILE = 128
TPE = CAP // TILE
NT = (E * CAP) // TILE
TRASH = E * CAP
XROWS = E * CAP + TILE

NWORK = 32
PPW = P // NWORK
TPW = T // NWORK
CH = 16


def _first_argmax(vals, iota, width):
    m = jnp.max(vals, axis=1, keepdims=True)
    eq = vals == m
    fidx = jnp.min(jnp.where(eq, iota, width), axis=1, keepdims=True)
    sel = iota == fidx
    return fidx, sel


def _route_body(x_ref, gw_ref, b_ref, sg_ref, su_ref, sd_ref,
                lo_ref, tk_ref, de_ref, wa_ref, wb_ref, cnt_ref, sh_ref,
                run_ref):
    step = pl.program_id(0)

    @pl.when(step == 0)
    def _():
        run_ref[...] = jnp.zeros_like(run_ref)

    x = x_ref[...]
    logits = lax.dot_general(x, gw_ref[...], (((1,), (1,)), ((), ())),
                             preferred_element_type=jnp.float32)
    lo_ref[...] = logits
    scores = 1.0 / (1.0 + jnp.exp(-logits))
    s_rt = scores + b_ref[...]

    iota8 = lax.broadcasted_iota(jnp.int32, (TB, GSZ), 1)
    gcols = []
    for g in range(NG):
        blk = s_rt[:, g * GSZ:(g + 1) * GSZ]
        m1 = jnp.max(blk, axis=1, keepdims=True)
        _, sel = _first_argmax(blk, iota8, GSZ)
        m2 = jnp.max(jnp.where(sel, -jnp.inf, blk), axis=1, keepdims=True)
        gcols.append(m1 + m2)
    gs = jnp.concatenate(gcols, axis=1)

    iota_g = lax.broadcasted_iota(jnp.int32, (TB, NG), 1)
    gmask = jnp.zeros((TB, NG), jnp.float32)
    for _ in range(TKG):
        _, sel = _first_argmax(gs, iota_g, NG)
        gmask = jnp.where(sel, 1.0, gmask)
        gs = jnp.where(sel, -jnp.inf, gs)

    cols = []
    for g in range(NG):
        keep = gmask[:, g:g + 1] > 0
        cols.append(jnp.where(keep, s_rt[:, g * GSZ:(g + 1) * GSZ], -jnp.inf))
    masked = jnp.concatenate(cols, axis=1)

    iota_e = lax.broadcasted_iota(jnp.int32, (TB, E), 1)
    idxs, scs = [], []
    for _ in range(TOPK):
        fidx, sel = _first_argmax(masked, iota_e, E)
        idxs.append(fidx)
        scs.append(jnp.sum(jnp.where(sel, scores, 0.0), axis=1, keepdims=True))
        masked = jnp.where(sel, -jnp.inf, masked)
    topk = jnp.concatenate(idxs, axis=1)
    tk_ref[...] = topk
    sc = jnp.concatenate(scs, axis=1)
    tw = sc / (jnp.sum(sc, axis=1, keepdims=True) + 1e-20)

    e0, e1 = idxs[0], idxs[1]
    oh0 = (iota_e == e0).astype(jnp.float32)
    oh1 = (iota_e == e1).astype(jnp.float32)
    ohsum = oh0 + oh1
    r_iota = lax.broadcasted_iota(jnp.int32, (TB, TB), 0)
    c_iota = lax.broadcasted_iota(jnp.int32, (TB, TB), 1)
    tril = (r_iota > c_iota).astype(jnp.float32)
    prev = jnp.dot(tril, ohsum, preferred_element_type=jnp.float32)
    prev = prev + run_ref[...]
    rank0 = jnp.sum(prev * oh0, axis=1, keepdims=True).astype(jnp.int32)
    rank1 = jnp.sum(prev * oh1, axis=1, keepdims=True).astype(jnp.int32)
    run_ref[...] = run_ref[...] + jnp.sum(ohsum, axis=0, keepdims=True)

    v0 = rank0 < CAP
    v1 = rank1 < CAP
    dest0 = jnp.where(v0, e0 * CAP + rank0, TRASH)
    dest1 = jnp.where(v1, e1 * CAP + rank1, TRASH)
    de_ref[...] = jnp.concatenate([dest0, dest1], axis=1)
    w0 = tw[:, 0:1] * v0.astype(jnp.float32)
    w1 = tw[:, 1:2] * v1.astype(jnp.float32)
    wa_ref[...] = jnp.broadcast_to(w0, (TB, 16))
    wb_ref[...] = jnp.broadcast_to(w1, (TB, 16))
    cnt_ref[...] = jnp.broadcast_to(run_ref[...], cnt_ref.shape).astype(jnp.int32)

    g_sh = jnp.dot(x, sg_ref[...], preferred_element_type=jnp.float32)
    u_sh = jnp.dot(x, su_ref[...], preferred_element_type=jnp.float32)
    h_sh = (g_sh * (1.0 / (1.0 + jnp.exp(-g_sh)))) * u_sh
    sh_ref[...] = jnp.dot(h_sh, sd_ref[...], preferred_element_type=jnp.float32)


def _route_call(x2d, gw, bias, Sg, Su, Sd):
    return pl.pallas_call(
        _route_body,
        grid=(NTB,),
        in_specs=[
            pl.BlockSpec((TB, D), lambda i: (i, 0)),
            pl.BlockSpec((E, D), lambda i: (0, 0)),
            pl.BlockSpec((1, E), lambda i: (0, 0)),
            pl.BlockSpec((D, FF), lambda i: (0, 0)),
            pl.BlockSpec((D, FF), lambda i: (0, 0)),
            pl.BlockSpec((FF, D), lambda i: (0, 0)),
        ],
        out_specs=[
            pl.BlockSpec((TB, E), lambda i: (i, 0)),
            pl.BlockSpec((TB, TOPK), lambda i: (i, 0)),
            pl.BlockSpec((TB, TOPK), lambda i: (i, 0)),
            pl.BlockSpec((TB, 16), lambda i: (i, 0)),
            pl.BlockSpec((TB, 16), lambda i: (i, 0)),
            pl.BlockSpec((8, E), lambda i: (0, 0)),
            pl.BlockSpec((TB, D), lambda i: (i, 0)),
        ],
        out_shape=[
            jax.ShapeDtypeStruct((T, E), jnp.float32),
            jax.ShapeDtypeStruct((T, TOPK), jnp.int32),
            jax.ShapeDtypeStruct((T, TOPK), jnp.int32),
            jax.ShapeDtypeStruct((T, 16), jnp.float32),
            jax.ShapeDtypeStruct((T, 16), jnp.float32),
            jax.ShapeDtypeStruct((8, E), jnp.int32),
            jax.ShapeDtypeStruct((T, D), jnp.float32),
        ],
        scratch_shapes=[pltpu.VMEM((1, E), jnp.float32)],
    )(x2d, gw, bias, Sg, Su, Sd)


def _dispatch_body(x_hbm, dest_hbm, xbuf_hbm,
                   dvm0, dvm1, rows0, rows1, sg0, sg1, ss0, ss1):
    wid = lax.axis_index("s") * 2 + lax.axis_index("c")
    lane = lax.iota(jnp.int32, 16)
    nch = PPW // CH
    dvms = [dvm0, dvm1]
    rows = [rows0, rows1]
    gsem = [sg0, sg1]
    ssem = [ss0, ss1]

    def issue_gather(c):
        base = wid * PPW + c * CH
        tok = (base + lane) & (T - 1)
        pltpu.sync_copy(dest_hbm.at[pl.ds(base, CH)], dvms[c % 2])
        return pltpu.async_copy(x_hbm.at[tok], rows[c % 2], gsem[c % 2])

    gh = [issue_gather(0), None]
    sh_ = [None, None]
    for c in range(nch):
        b = c % 2
        nb = (c + 1) % 2
        if c + 1 < nch:
            if sh_[nb] is not None:
                sh_[nb].wait()
            gh[nb] = issue_gather(c + 1)
        gh[b].wait()
        sh_[b] = pltpu.async_copy(rows[b], xbuf_hbm.at[dvms[b][...]], ssem[b])
    sh_[(nch - 1) % 2].wait()
    if sh_[nch % 2] is not None:
        sh_[nch % 2].wait()


def _dispatch_call(x2d, dest):
    mesh = plsc.VectorSubcoreMesh(core_axis_name="c", subcore_axis_name="s")
    f = functools.partial(
        pl.kernel,
        out_type=jax.ShapeDtypeStruct((XROWS, D), jnp.float32),
        mesh=mesh,
        scratch_types=[
            pltpu.VMEM((CH,), jnp.int32),
            pltpu.VMEM((CH,), jnp.int32),
            pltpu.VMEM((CH, D), jnp.float32),
            pltpu.VMEM((CH, D), jnp.float32),
            pltpu.SemaphoreType.DMA,
            pltpu.SemaphoreType.DMA,
            pltpu.SemaphoreType.DMA,
            pltpu.SemaphoreType.DMA,
        ],
    )(_dispatch_body)
    return f(x2d, dest)


def _ffn_body(cnt_ref, x_ref, wg_ref, wu_ref, wd_ref, o_ref):
    g = pl.program_id(0)
    e = jnp.minimum(g // TPE, E - 1)
    t = g % TPE
    is_trash = g >= NT
    active = jnp.logical_and(g < NT, cnt_ref[e] > t * TILE)

    @pl.when(is_trash)
    def _():
        o_ref[...] = jnp.zeros_like(o_ref)

    @pl.when(active)
    def _():
        x = x_ref[...]
        gg = jnp.dot(x, wg_ref[0], preferred_element_type=jnp.float32)
        uu = jnp.dot(x, wu_ref[0], preferred_element_type=jnp.float32)
        h = (gg * (1.0 / (1.0 + jnp.exp(-gg)))) * uu
        o_ref[...] = jnp.dot(h, wd_ref[0], preferred_element_type=jnp.float32)


def _ffn_xmap(g, cnt):
    e = jnp.minimum(g // TPE, E - 1)
    t = g % TPE
    last_act = jnp.maximum((cnt[e] + TILE - 1) // TILE - 1, 0)
    b = jnp.where(g >= NT, NT, TPE * e + jnp.minimum(t, last_act))
    return (b, 0)


def _ffn_call(counts, xbuf, Wg, Wu, Wd):
    grid_spec = pltpu.PrefetchScalarGridSpec(
        num_scalar_prefetch=1,
        grid=(NT + 1,),
        in_specs=[
            pl.BlockSpec((TILE, D), _ffn_xmap),
            pl.BlockSpec((1, D, FF), lambda g, c: (jnp.minimum(g // TPE, E - 1), 0, 0)),
            pl.BlockSpec((1, D, FF), lambda g, c: (jnp.minimum(g // TPE, E - 1), 0, 0)),
            pl.BlockSpec((1, FF, D), lambda g, c: (jnp.minimum(g // TPE, E - 1), 0, 0)),
        ],
        out_specs=pl.BlockSpec((TILE, D), _ffn_xmap),
    )
    return pl.pallas_call(
        _ffn_body,
        grid_spec=grid_spec,
        out_shape=jax.ShapeDtypeStruct((XROWS, D), jnp.float32),
    )(counts, xbuf, Wg, Wu, Wd)


def _combine_body(osort_hbm, dest_hbm, wa_hbm, wb_hbm, sh_hbm, y_hbm,
                  dva0, dva1, dvb0, dvb1, wva0, wva1, wvb0, wvb1,
                  ra0, ra1, rb0, rb1, rs0, rs1,
                  sa0, sa1, sb0, sb1, ss0, ss1, so0, so1):
    wid = lax.axis_index("s") * 2 + lax.axis_index("c")
    nch = TPW // CH
    dva = [dva0, dva1]
    dvb = [dvb0, dvb1]
    wva = [wva0, wva1]
    wvb = [wvb0, wvb1]
    ra = [ra0, ra1]
    rb = [rb0, rb1]
    rs = [rs0, rs1]
    sa = [sa0, sa1]
    sb = [sb0, sb1]
    ss = [ss0, ss1]
    so = [so0, so1]

    def issue(c):
        t0 = wid * TPW + c * CH
        b = c % 2
        pltpu.sync_copy(dest_hbm.at[pl.ds(t0, CH)], dva[b])
        pltpu.sync_copy(dest_hbm.at[pl.ds(T + t0, CH)], dvb[b])
        pltpu.sync_copy(wa_hbm.at[pl.ds(t0, CH)], wva[b])
        pltpu.sync_copy(wb_hbm.at[pl.ds(t0, CH)], wvb[b])
        return (pltpu.async_copy(osort_hbm.at[dva[b][...]], ra[b], sa[b]),
                pltpu.async_copy(osort_hbm.at[dvb[b][...]], rb[b], sb[b]),
                pltpu.async_copy(sh_hbm.at[pl.ds(t0, CH)], rs[b], ss[b]))

    hnd = [issue(0), None]
    out_h = [None, None]
    for c in range(nch):
        b = c % 2
        nb = (c + 1) % 2
        if c + 1 < nch:
            if out_h[nb] is not None:
                out_h[nb].wait()
            hnd[nb] = issue(c + 1)
        for h in hnd[b]:
            h.wait()
        for r in range(CH):
            wa = wva[b][r]
            wb = wvb[b][r]

            def body(j, _):
                sl = pl.ds(j * 16, 16)
                ra[b][r, sl] = wa * ra[b][r, sl] + wb * rb[b][r, sl] + rs[b][r, sl]
                return 0
            lax.fori_loop(0, D // 16, body, 0)
        t0 = wid * TPW + c * CH
        out_h[b] = pltpu.async_copy(ra[b], y_hbm.at[pl.ds(t0, CH)], so[b])
    out_h[(nch - 1) % 2].wait()
    if out_h[nch % 2] is not None:
        out_h[nch % 2].wait()


def _combine_call(osort, dest, wba, wbb, sh):
    mesh = plsc.VectorSubcoreMesh(core_axis_name="c", subcore_axis_name="s")
    f = functools.partial(
        pl.kernel,
        out_type=jax.ShapeDtypeStruct((T, D), jnp.float32),
        mesh=mesh,
        scratch_types=(
            [pltpu.VMEM((CH,), jnp.int32)] * 4
            + [pltpu.VMEM((CH, 16), jnp.float32)] * 4
            + [pltpu.VMEM((CH, D), jnp.float32)] * 6
            + [pltpu.SemaphoreType.DMA] * 8
        ),
    )(_combine_body)
    return f(osort, dest, wba, wbb, sh)


def kernel(hidden_states, gate_weight, expert_bias, Wg, Wu, Wd, Sg, Su, Sd):
    x2d = hidden_states.reshape(T, D)
    logits, topk, dest2d, wba, wbb, counts8, sh = _route_call(
        x2d, gate_weight, expert_bias.reshape(1, E), Sg, Su, Sd)
    dest = jnp.transpose(dest2d).reshape(P)
    counts = counts8[0]
    xbuf = _dispatch_call(x2d, dest)
    osort = _ffn_call(counts, xbuf, Wg, Wu, Wd)
    y = _combine_call(osort, dest, wba, wbb, sh)
    return (y.reshape(B, S, D), logits.reshape(B, S, E),
            topk.reshape(B, S, TOPK))


# speedup vs baseline: 1.0946x; 1.0246x over previous
import functools

import jax
import jax.numpy as jnp
from jax import lax
from jax.experimental import pallas as pl
from jax.experimental.pallas import tpu as pltpu
from jax.experimental.pallas import tpu_sc as plsc

E = 64
TOPK = 2
NG = 8
GSZ = E // NG
TKG = 4
D = 1024
FF = 256
CAP = 256
B = 1
S = 2048
T = B * S
P = T * TOPK

TB = 1024
NTB = T // TB

TILE = 128
TPE = CAP // TILE
NT = (E * CAP) // TILE
TRASH = E * CAP
XROWS = E * CAP + TILE

NWORK = 32
PPW = P // NWORK
TPW = T // NWORK
CH = 16


def _first_argmax(vals, iota, width):
    m = jnp.max(vals, axis=1, keepdims=True)
    eq = vals == m
    fidx = jnp.min(jnp.where(eq, iota, width), axis=1, keepdims=True)
    sel = iota == fidx
    return fidx, sel


def _route_body(x_ref, gw_ref, b_ref,
                lo_ref, tk_ref, de_ref, wa_ref, wb_ref, cnt_ref,
                run_ref):
    step = pl.program_id(0)

    @pl.when(step == 0)
    def _():
        run_ref[...] = jnp.zeros_like(run_ref)

    x = x_ref[...]
    logits = lax.dot_general(x, gw_ref[...], (((1,), (1,)), ((), ())),
                             preferred_element_type=jnp.float32)
    lo_ref[...] = logits
    scores = 1.0 / (1.0 + jnp.exp(-logits))
    s_rt = scores + b_ref[...]

    iota8 = lax.broadcasted_iota(jnp.int32, (TB, GSZ), 1)
    gcols = []
    for g in range(NG):
        blk = s_rt[:, g * GSZ:(g + 1) * GSZ]
        m1 = jnp.max(blk, axis=1, keepdims=True)
        _, sel = _first_argmax(blk, iota8, GSZ)
        m2 = jnp.max(jnp.where(sel, -jnp.inf, blk), axis=1, keepdims=True)
        gcols.append(m1 + m2)
    gs = jnp.concatenate(gcols, axis=1)

    iota_g = lax.broadcasted_iota(jnp.int32, (TB, NG), 1)
    gmask = jnp.zeros((TB, NG), jnp.float32)
    for _ in range(TKG):
        _, sel = _first_argmax(gs, iota_g, NG)
        gmask = jnp.where(sel, 1.0, gmask)
        gs = jnp.where(sel, -jnp.inf, gs)

    cols = []
    for g in range(NG):
        keep = gmask[:, g:g + 1] > 0
        cols.append(jnp.where(keep, s_rt[:, g * GSZ:(g + 1) * GSZ], -jnp.inf))
    masked = jnp.concatenate(cols, axis=1)

    iota_e = lax.broadcasted_iota(jnp.int32, (TB, E), 1)
    idxs, scs = [], []
    for _ in range(TOPK):
        fidx, sel = _first_argmax(masked, iota_e, E)
        idxs.append(fidx)
        scs.append(jnp.sum(jnp.where(sel, scores, 0.0), axis=1, keepdims=True))
        masked = jnp.where(sel, -jnp.inf, masked)
    topk = jnp.concatenate(idxs, axis=1)
    tk_ref[...] = topk
    sc = jnp.concatenate(scs, axis=1)
    tw = sc / (jnp.sum(sc, axis=1, keepdims=True) + 1e-20)

    e0, e1 = idxs[0], idxs[1]
    oh0 = (iota_e == e0).astype(jnp.float32)
    oh1 = (iota_e == e1).astype(jnp.float32)
    ohsum = oh0 + oh1
    r_iota = lax.broadcasted_iota(jnp.int32, (TB, TB), 0)
    c_iota = lax.broadcasted_iota(jnp.int32, (TB, TB), 1)
    tril = (r_iota > c_iota).astype(jnp.float32)
    prev = jnp.dot(tril, ohsum, preferred_element_type=jnp.float32)
    prev = prev + run_ref[...]
    rank0 = jnp.sum(prev * oh0, axis=1, keepdims=True).astype(jnp.int32)
    rank1 = jnp.sum(prev * oh1, axis=1, keepdims=True).astype(jnp.int32)
    run_ref[...] = run_ref[...] + jnp.sum(ohsum, axis=0, keepdims=True)

    v0 = rank0 < CAP
    v1 = rank1 < CAP
    dest0 = jnp.where(v0, e0 * CAP + rank0, TRASH)
    dest1 = jnp.where(v1, e1 * CAP + rank1, TRASH)
    de_ref[...] = jnp.concatenate([dest0, dest1], axis=1)
    w0 = tw[:, 0:1] * v0.astype(jnp.float32)
    w1 = tw[:, 1:2] * v1.astype(jnp.float32)
    wa_ref[...] = jnp.broadcast_to(w0, (TB, 16))
    wb_ref[...] = jnp.broadcast_to(w1, (TB, 16))
    cnt_ref[...] = jnp.broadcast_to(run_ref[...], cnt_ref.shape).astype(jnp.int32)


def _route_call(x2d, gw, bias):
    return pl.pallas_call(
        _route_body,
        grid=(NTB,),
        in_specs=[
            pl.BlockSpec((TB, D), lambda i: (i, 0)),
            pl.BlockSpec((E, D), lambda i: (0, 0)),
            pl.BlockSpec((1, E), lambda i: (0, 0)),
        ],
        out_specs=[
            pl.BlockSpec((TB, E), lambda i: (i, 0)),
            pl.BlockSpec((TB, TOPK), lambda i: (i, 0)),
            pl.BlockSpec((TB, TOPK), lambda i: (i, 0)),
            pl.BlockSpec((TB, 16), lambda i: (i, 0)),
            pl.BlockSpec((TB, 16), lambda i: (i, 0)),
            pl.BlockSpec((8, E), lambda i: (0, 0)),
        ],
        out_shape=[
            jax.ShapeDtypeStruct((T, E), jnp.float32),
            jax.ShapeDtypeStruct((T, TOPK), jnp.int32),
            jax.ShapeDtypeStruct((T, TOPK), jnp.int32),
            jax.ShapeDtypeStruct((T, 16), jnp.float32),
            jax.ShapeDtypeStruct((T, 16), jnp.float32),
            jax.ShapeDtypeStruct((8, E), jnp.int32),
        ],
        scratch_shapes=[pltpu.VMEM((1, E), jnp.float32)],
    )(x2d, gw, bias)


def _shared_body(x_ref, sg_ref, su_ref, sd_ref, sh_ref):
    x = x_ref[...]
    g_sh = jnp.dot(x, sg_ref[...], preferred_element_type=jnp.float32)
    u_sh = jnp.dot(x, su_ref[...], preferred_element_type=jnp.float32)
    h_sh = (g_sh * (1.0 / (1.0 + jnp.exp(-g_sh)))) * u_sh
    sh_ref[...] = jnp.dot(h_sh, sd_ref[...], preferred_element_type=jnp.float32)


def _shared_call(x2d, Sg, Su, Sd):
    return pl.pallas_call(
        _shared_body,
        grid=(NTB,),
        in_specs=[
            pl.BlockSpec((TB, D), lambda i: (i, 0)),
            pl.BlockSpec((D, FF), lambda i: (0, 0)),
            pl.BlockSpec((D, FF), lambda i: (0, 0)),
            pl.BlockSpec((FF, D), lambda i: (0, 0)),
        ],
        out_specs=pl.BlockSpec((TB, D), lambda i: (i, 0)),
        out_shape=jax.ShapeDtypeStruct((T, D), jnp.float32),
    )(x2d, Sg, Su, Sd)


def _dispatch_body(x_hbm, dest_hbm, xbuf_hbm,
                   dvm0, dvm1, rows0, rows1, sg0, sg1, ss0, ss1):
    wid = lax.axis_index("s") * 2 + lax.axis_index("c")
    lane = lax.iota(jnp.int32, 16)
    nch = PPW // CH
    dvms = [dvm0, dvm1]
    rows = [rows0, rows1]
    gsem = [sg0, sg1]
    ssem = [ss0, ss1]

    def issue_gather(c):
        base = wid * PPW + c * CH
        tok = (base + lane) & (T - 1)
        pltpu.sync_copy(dest_hbm.at[pl.ds(base, CH)], dvms[c % 2])
        return pltpu.async_copy(x_hbm.at[tok], rows[c % 2], gsem[c % 2])

    gh = [issue_gather(0), None]
    sh_ = [None, None]
    for c in range(nch):
        b = c % 2
        nb = (c + 1) % 2
        if c + 1 < nch:
            if sh_[nb] is not None:
                sh_[nb].wait()
            gh[nb] = issue_gather(c + 1)
        gh[b].wait()
        sh_[b] = pltpu.async_copy(rows[b], xbuf_hbm.at[dvms[b][...]], ssem[b])
    sh_[(nch - 1) % 2].wait()
    if sh_[nch % 2] is not None:
        sh_[nch % 2].wait()


def _dispatch_call(x2d, dest):
    mesh = plsc.VectorSubcoreMesh(core_axis_name="c", subcore_axis_name="s")
    f = functools.partial(
        pl.kernel,
        out_type=jax.ShapeDtypeStruct((XROWS, D), jnp.float32),
        mesh=mesh,
        scratch_types=[
            pltpu.VMEM((CH,), jnp.int32),
            pltpu.VMEM((CH,), jnp.int32),
            pltpu.VMEM((CH, D), jnp.float32),
            pltpu.VMEM((CH, D), jnp.float32),
            pltpu.SemaphoreType.DMA,
            pltpu.SemaphoreType.DMA,
            pltpu.SemaphoreType.DMA,
            pltpu.SemaphoreType.DMA,
        ],
    )(_dispatch_body)
    return f(x2d, dest)


def _ffn_body(cnt_ref, x_ref, wg_ref, wu_ref, wd_ref, o_ref):
    g = pl.program_id(0)
    e = jnp.minimum(g // TPE, E - 1)
    t = g % TPE
    is_trash = g >= NT
    active = jnp.logical_and(g < NT, cnt_ref[e] > t * TILE)

    @pl.when(is_trash)
    def _():
        o_ref[...] = jnp.zeros_like(o_ref)

    @pl.when(active)
    def _():
        x = x_ref[...]
        gg = jnp.dot(x, wg_ref[0], preferred_element_type=jnp.float32)
        uu = jnp.dot(x, wu_ref[0], preferred_element_type=jnp.float32)
        h = (gg * (1.0 / (1.0 + jnp.exp(-gg)))) * uu
        o_ref[...] = jnp.dot(h, wd_ref[0], preferred_element_type=jnp.float32)


def _ffn_xmap(g, cnt):
    e = jnp.minimum(g // TPE, E - 1)
    t = g % TPE
    last_act = jnp.maximum((cnt[e] + TILE - 1) // TILE - 1, 0)
    b = jnp.where(g >= NT, NT, TPE * e + jnp.minimum(t, last_act))
    return (b, 0)


def _ffn_call(counts, xbuf, Wg, Wu, Wd):
    grid_spec = pltpu.PrefetchScalarGridSpec(
        num_scalar_prefetch=1,
        grid=(NT + 1,),
        in_specs=[
            pl.BlockSpec((TILE, D), _ffn_xmap),
            pl.BlockSpec((1, D, FF), lambda g, c: (jnp.minimum(g // TPE, E - 1), 0, 0)),
            pl.BlockSpec((1, D, FF), lambda g, c: (jnp.minimum(g // TPE, E - 1), 0, 0)),
            pl.BlockSpec((1, FF, D), lambda g, c: (jnp.minimum(g // TPE, E - 1), 0, 0)),
        ],
        out_specs=pl.BlockSpec((TILE, D), _ffn_xmap),
    )
    return pl.pallas_call(
        _ffn_body,
        grid_spec=grid_spec,
        out_shape=jax.ShapeDtypeStruct((XROWS, D), jnp.float32),
    )(counts, xbuf, Wg, Wu, Wd)


def _combine_body(osort_hbm, dest_hbm, wa_hbm, wb_hbm, sh_hbm, y_hbm,
                  dva0, dva1, dvb0, dvb1, wva0, wva1, wvb0, wvb1,
                  ra0, ra1, rb0, rb1, rs0, rs1,
                  sa0, sa1, sb0, sb1, ss0, ss1, so0, so1):
    wid = lax.axis_index("s") * 2 + lax.axis_index("c")
    nch = TPW // CH
    dva = [dva0, dva1]
    dvb = [dvb0, dvb1]
    wva = [wva0, wva1]
    wvb = [wvb0, wvb1]
    ra = [ra0, ra1]
    rb = [rb0, rb1]
    rs = [rs0, rs1]
    sa = [sa0, sa1]
    sb = [sb0, sb1]
    ss = [ss0, ss1]
    so = [so0, so1]

    def issue(c):
        t0 = wid * TPW + c * CH
        b = c % 2
        pltpu.sync_copy(dest_hbm.at[pl.ds(t0, CH)], dva[b])
        pltpu.sync_copy(dest_hbm.at[pl.ds(T + t0, CH)], dvb[b])
        pltpu.sync_copy(wa_hbm.at[pl.ds(t0, CH)], wva[b])
        pltpu.sync_copy(wb_hbm.at[pl.ds(t0, CH)], wvb[b])
        return (pltpu.async_copy(osort_hbm.at[dva[b][...]], ra[b], sa[b]),
                pltpu.async_copy(osort_hbm.at[dvb[b][...]], rb[b], sb[b]),
                pltpu.async_copy(sh_hbm.at[pl.ds(t0, CH)], rs[b], ss[b]))

    hnd = [issue(0), None]
    out_h = [None, None]
    for c in range(nch):
        b = c % 2
        nb = (c + 1) % 2
        if c + 1 < nch:
            if out_h[nb] is not None:
                out_h[nb].wait()
            hnd[nb] = issue(c + 1)
        for h in hnd[b]:
            h.wait()
        for r in range(CH):
            wa = wva[b][r]
            wb = wvb[b][r]

            def body(j, _):
                for u in range(4):
                    sl = pl.ds(j * 64 + u * 16, 16)
                    ra[b][r, sl] = (wa * ra[b][r, sl] + wb * rb[b][r, sl]
                                    + rs[b][r, sl])
                return 0
            lax.fori_loop(0, D // 64, body, 0)
        t0 = wid * TPW + c * CH
        out_h[b] = pltpu.async_copy(ra[b], y_hbm.at[pl.ds(t0, CH)], so[b])
    out_h[(nch - 1) % 2].wait()
    if out_h[nch % 2] is not None:
        out_h[nch % 2].wait()


def _combine_call(osort, dest, wba, wbb, sh):
    mesh = plsc.VectorSubcoreMesh(core_axis_name="c", subcore_axis_name="s")
    f = functools.partial(
        pl.kernel,
        out_type=jax.ShapeDtypeStruct((T, D), jnp.float32),
        mesh=mesh,
        scratch_types=(
            [pltpu.VMEM((CH,), jnp.int32)] * 4
            + [pltpu.VMEM((CH, 16), jnp.float32)] * 4
            + [pltpu.VMEM((CH, D), jnp.float32)] * 6
            + [pltpu.SemaphoreType.DMA] * 8
        ),
    )(_combine_body)
    return f(osort, dest, wba, wbb, sh)


def kernel(hidden_states, gate_weight, expert_bias, Wg, Wu, Wd, Sg, Su, Sd):
    x2d = hidden_states.reshape(T, D)
    logits, topk, dest2d, wba, wbb, counts8 = _route_call(
        x2d, gate_weight, expert_bias.reshape(1, E))
    dest = jnp.transpose(dest2d).reshape(P)
    counts = counts8[0]
    xbuf = _dispatch_call(x2d, dest)
    sh = _shared_call(x2d, Sg, Su, Sd)
    osort = _ffn_call(counts, xbuf, Wg, Wu, Wd)
    y = _combine_call(osort, dest, wba, wbb, sh)
    return (y.reshape(B, S, D), logits.reshape(B, S, E),
            topk.reshape(B, S, TOPK))


# speedup vs baseline: 1.0952x; 1.0006x over previous
import functools

import jax
import jax.numpy as jnp
from jax import lax
from jax.experimental import pallas as pl
from jax.experimental.pallas import tpu as pltpu
from jax.experimental.pallas import tpu_sc as plsc

E = 64
TOPK = 2
NG = 8
GSZ = E // NG
TKG = 4
D = 1024
FF = 256
CAP = 256
B = 1
S = 2048
T = B * S
P = T * TOPK

TB = 1024
NTB = T // TB

TILE = 128
TPE = CAP // TILE
NT = (E * CAP) // TILE
TRASH = E * CAP
XROWS = E * CAP + TILE

NWORK = 32
PPW = P // NWORK
TPW = T // NWORK
CH = 16


def _first_argmax(vals, iota, width):
    m = jnp.max(vals, axis=1, keepdims=True)
    eq = vals == m
    fidx = jnp.min(jnp.where(eq, iota, width), axis=1, keepdims=True)
    sel = iota == fidx
    return fidx, sel


def _route_body(x_ref, gw_ref, b_ref,
                lo_ref, tk_ref, de_ref, wa_ref, wb_ref, cnt_ref,
                run_ref, tril_ref):
    step = pl.program_id(0)

    @pl.when(step == 0)
    def _():
        run_ref[...] = jnp.zeros_like(run_ref)
        r_iota = lax.broadcasted_iota(jnp.int32, (TB, TB), 0)
        c_iota = lax.broadcasted_iota(jnp.int32, (TB, TB), 1)
        tril_ref[...] = (r_iota > c_iota).astype(jnp.float32)

    x = x_ref[...]
    logits = lax.dot_general(x, gw_ref[...], (((1,), (1,)), ((), ())),
                             preferred_element_type=jnp.float32)
    lo_ref[...] = logits
    scores = 1.0 / (1.0 + jnp.exp(-logits))
    s_rt = scores + b_ref[...]

    iota8 = lax.broadcasted_iota(jnp.int32, (TB, GSZ), 1)
    gcols = []
    for g in range(NG):
        blk = s_rt[:, g * GSZ:(g + 1) * GSZ]
        m1 = jnp.max(blk, axis=1, keepdims=True)
        _, sel = _first_argmax(blk, iota8, GSZ)
        m2 = jnp.max(jnp.where(sel, -jnp.inf, blk), axis=1, keepdims=True)
        gcols.append(m1 + m2)
    gs = jnp.concatenate(gcols, axis=1)

    iota_g = lax.broadcasted_iota(jnp.int32, (TB, NG), 1)
    gmask = jnp.zeros((TB, NG), jnp.float32)
    for _ in range(TKG):
        _, sel = _first_argmax(gs, iota_g, NG)
        gmask = jnp.where(sel, 1.0, gmask)
        gs = jnp.where(sel, -jnp.inf, gs)

    cols = []
    for g in range(NG):
        keep = gmask[:, g:g + 1] > 0
        cols.append(jnp.where(keep, s_rt[:, g * GSZ:(g + 1) * GSZ], -jnp.inf))
    masked = jnp.concatenate(cols, axis=1)

    iota_e = lax.broadcasted_iota(jnp.int32, (TB, E), 1)
    idxs, scs = [], []
    for _ in range(TOPK):
        fidx, sel = _first_argmax(masked, iota_e, E)
        idxs.append(fidx)
        scs.append(jnp.sum(jnp.where(sel, scores, 0.0), axis=1, keepdims=True))
        masked = jnp.where(sel, -jnp.inf, masked)
    topk = jnp.concatenate(idxs, axis=1)
    tk_ref[...] = topk
    sc = jnp.concatenate(scs, axis=1)
    tw = sc / (jnp.sum(sc, axis=1, keepdims=True) + 1e-20)

    e0, e1 = idxs[0], idxs[1]
    oh0 = (iota_e == e0).astype(jnp.float32)
    oh1 = (iota_e == e1).astype(jnp.float32)
    ohsum = oh0 + oh1
    prev = jnp.dot(tril_ref[...], ohsum, preferred_element_type=jnp.float32)
    prev = prev + run_ref[...]
    rank0 = jnp.sum(prev * oh0, axis=1, keepdims=True).astype(jnp.int32)
    rank1 = jnp.sum(prev * oh1, axis=1, keepdims=True).astype(jnp.int32)
    run_ref[...] = run_ref[...] + jnp.sum(ohsum, axis=0, keepdims=True)

    v0 = rank0 < CAP
    v1 = rank1 < CAP
    dest0 = jnp.where(v0, e0 * CAP + rank0, TRASH)
    dest1 = jnp.where(v1, e1 * CAP + rank1, TRASH)
    de_ref[...] = jnp.concatenate([dest0, dest1], axis=1)
    w0 = tw[:, 0:1] * v0.astype(jnp.float32)
    w1 = tw[:, 1:2] * v1.astype(jnp.float32)
    wa_ref[...] = jnp.broadcast_to(w0, (TB, 16))
    wb_ref[...] = jnp.broadcast_to(w1, (TB, 16))
    cnt_ref[...] = jnp.broadcast_to(run_ref[...], cnt_ref.shape).astype(jnp.int32)


def _route_call(x2d, gw, bias):
    return pl.pallas_call(
        _route_body,
        grid=(NTB,),
        in_specs=[
            pl.BlockSpec((TB, D), lambda i: (i, 0)),
            pl.BlockSpec((E, D), lambda i: (0, 0)),
            pl.BlockSpec((1, E), lambda i: (0, 0)),
        ],
        out_specs=[
            pl.BlockSpec((TB, E), lambda i: (i, 0)),
            pl.BlockSpec((TB, TOPK), lambda i: (i, 0)),
            pl.BlockSpec((TB, TOPK), lambda i: (i, 0)),
            pl.BlockSpec((TB, 16), lambda i: (i, 0)),
            pl.BlockSpec((TB, 16), lambda i: (i, 0)),
            pl.BlockSpec((8, E), lambda i: (0, 0)),
        ],
        out_shape=[
            jax.ShapeDtypeStruct((T, E), jnp.float32),
            jax.ShapeDtypeStruct((T, TOPK), jnp.int32),
            jax.ShapeDtypeStruct((T, TOPK), jnp.int32),
            jax.ShapeDtypeStruct((T, 16), jnp.float32),
            jax.ShapeDtypeStruct((T, 16), jnp.float32),
            jax.ShapeDtypeStruct((8, E), jnp.int32),
        ],
        scratch_shapes=[pltpu.VMEM((1, E), jnp.float32),
                        pltpu.VMEM((TB, TB), jnp.float32)],
    )(x2d, gw, bias)


def _shared_body(x_ref, sg_ref, su_ref, sd_ref, sh_ref):
    x = x_ref[...]
    g_sh = jnp.dot(x, sg_ref[...], preferred_element_type=jnp.float32)
    u_sh = jnp.dot(x, su_ref[...], preferred_element_type=jnp.float32)
    h_sh = (g_sh * (1.0 / (1.0 + jnp.exp(-g_sh)))) * u_sh
    sh_ref[...] = jnp.dot(h_sh, sd_ref[...], preferred_element_type=jnp.float32)


def _shared_call(x2d, Sg, Su, Sd):
    return pl.pallas_call(
        _shared_body,
        grid=(NTB,),
        in_specs=[
            pl.BlockSpec((TB, D), lambda i: (i, 0)),
            pl.BlockSpec((D, FF), lambda i: (0, 0)),
            pl.BlockSpec((D, FF), lambda i: (0, 0)),
            pl.BlockSpec((FF, D), lambda i: (0, 0)),
        ],
        out_specs=pl.BlockSpec((TB, D), lambda i: (i, 0)),
        out_shape=jax.ShapeDtypeStruct((T, D), jnp.float32),
    )(x2d, Sg, Su, Sd)


def _dispatch_body(x_hbm, dest_hbm, xbuf_hbm,
                   dvm0, dvm1, rows0, rows1, sg0, sg1, ss0, ss1):
    wid = lax.axis_index("s") * 2 + lax.axis_index("c")
    lane = lax.iota(jnp.int32, 16)
    nch = PPW // CH
    dvms = [dvm0, dvm1]
    rows = [rows0, rows1]
    gsem = [sg0, sg1]
    ssem = [ss0, ss1]

    def issue_gather(c):
        base = wid * PPW + c * CH
        tok = (base + lane) & (T - 1)
        pltpu.sync_copy(dest_hbm.at[pl.ds(base, CH)], dvms[c % 2])
        return pltpu.async_copy(x_hbm.at[tok], rows[c % 2], gsem[c % 2])

    gh = [issue_gather(0), None]
    sh_ = [None, None]
    for c in range(nch):
        b = c % 2
        nb = (c + 1) % 2
        if c + 1 < nch:
            if sh_[nb] is not None:
                sh_[nb].wait()
            gh[nb] = issue_gather(c + 1)
        gh[b].wait()
        sh_[b] = pltpu.async_copy(rows[b], xbuf_hbm.at[dvms[b][...]], ssem[b])
    sh_[(nch - 1) % 2].wait()
    if sh_[nch % 2] is not None:
        sh_[nch % 2].wait()


def _dispatch_call(x2d, dest):
    mesh = plsc.VectorSubcoreMesh(core_axis_name="c", subcore_axis_name="s")
    f = functools.partial(
        pl.kernel,
        out_type=jax.ShapeDtypeStruct((XROWS, D), jnp.float32),
        mesh=mesh,
        scratch_types=[
            pltpu.VMEM((CH,), jnp.int32),
            pltpu.VMEM((CH,), jnp.int32),
            pltpu.VMEM((CH, D), jnp.float32),
            pltpu.VMEM((CH, D), jnp.float32),
            pltpu.SemaphoreType.DMA,
            pltpu.SemaphoreType.DMA,
            pltpu.SemaphoreType.DMA,
            pltpu.SemaphoreType.DMA,
        ],
    )(_dispatch_body)
    return f(x2d, dest)


def _ffn_body(cnt_ref, x_ref, wg_ref, wu_ref, wd_ref, o_ref):
    g = pl.program_id(0)
    e = jnp.minimum(g // TPE, E - 1)
    t = g % TPE
    is_trash = g >= NT
    active = jnp.logical_and(g < NT, cnt_ref[e] > t * TILE)

    @pl.when(is_trash)
    def _():
        o_ref[...] = jnp.zeros_like(o_ref)

    @pl.when(active)
    def _():
        x = x_ref[...]
        gg = jnp.dot(x, wg_ref[0], preferred_element_type=jnp.float32)
        uu = jnp.dot(x, wu_ref[0], preferred_element_type=jnp.float32)
        h = (gg * (1.0 / (1.0 + jnp.exp(-gg)))) * uu
        o_ref[...] = jnp.dot(h, wd_ref[0], preferred_element_type=jnp.float32)


def _ffn_xmap(g, cnt):
    e = jnp.minimum(g // TPE, E - 1)
    t = g % TPE
    last_act = jnp.maximum((cnt[e] + TILE - 1) // TILE - 1, 0)
    b = jnp.where(g >= NT, NT, TPE * e + jnp.minimum(t, last_act))
    return (b, 0)


def _ffn_call(counts, xbuf, Wg, Wu, Wd):
    grid_spec = pltpu.PrefetchScalarGridSpec(
        num_scalar_prefetch=1,
        grid=(NT + 1,),
        in_specs=[
            pl.BlockSpec((TILE, D), _ffn_xmap),
            pl.BlockSpec((1, D, FF), lambda g, c: (jnp.minimum(g // TPE, E - 1), 0, 0)),
            pl.BlockSpec((1, D, FF), lambda g, c: (jnp.minimum(g // TPE, E - 1), 0, 0)),
            pl.BlockSpec((1, FF, D), lambda g, c: (jnp.minimum(g // TPE, E - 1), 0, 0)),
        ],
        out_specs=pl.BlockSpec((TILE, D), _ffn_xmap),
    )
    return pl.pallas_call(
        _ffn_body,
        grid_spec=grid_spec,
        out_shape=jax.ShapeDtypeStruct((XROWS, D), jnp.float32),
    )(counts, xbuf, Wg, Wu, Wd)


def _combine_body(osort_hbm, dest_hbm, wa_hbm, wb_hbm, sh_hbm, y_hbm,
                  dva0, dva1, dvb0, dvb1, wva0, wva1, wvb0, wvb1,
                  ra0, ra1, rb0, rb1, rs0, rs1,
                  sa0, sa1, sb0, sb1, ss0, ss1, so0, so1):
    wid = lax.axis_index("s") * 2 + lax.axis_index("c")
    nch = TPW // CH
    dva = [dva0, dva1]
    dvb = [dvb0, dvb1]
    wva = [wva0, wva1]
    wvb = [wvb0, wvb1]
    ra = [ra0, ra1]
    rb = [rb0, rb1]
    rs = [rs0, rs1]
    sa = [sa0, sa1]
    sb = [sb0, sb1]
    ss = [ss0, ss1]
    so = [so0, so1]

    def issue(c):
        t0 = wid * TPW + c * CH
        b = c % 2
        pltpu.sync_copy(dest_hbm.at[pl.ds(t0, CH)], dva[b])
        pltpu.sync_copy(dest_hbm.at[pl.ds(T + t0, CH)], dvb[b])
        pltpu.sync_copy(wa_hbm.at[pl.ds(t0, CH)], wva[b])
        pltpu.sync_copy(wb_hbm.at[pl.ds(t0, CH)], wvb[b])
        return (pltpu.async_copy(osort_hbm.at[dva[b][...]], ra[b], sa[b]),
                pltpu.async_copy(osort_hbm.at[dvb[b][...]], rb[b], sb[b]),
                pltpu.async_copy(sh_hbm.at[pl.ds(t0, CH)], rs[b], ss[b]))

    hnd = [issue(0), None]
    out_h = [None, None]
    for c in range(nch):
        b = c % 2
        nb = (c + 1) % 2
        if c + 1 < nch:
            if out_h[nb] is not None:
                out_h[nb].wait()
            hnd[nb] = issue(c + 1)
        for h in hnd[b]:
            h.wait()
        for r in range(CH):
            wa = wva[b][r]
            wb = wvb[b][r]

            def body(j, _):
                for u in range(4):
                    sl = pl.ds(j * 64 + u * 16, 16)
                    ra[b][r, sl] = (wa * ra[b][r, sl] + wb * rb[b][r, sl]
                                    + rs[b][r, sl])
                return 0
            lax.fori_loop(0, D // 64, body, 0)
        t0 = wid * TPW + c * CH
        out_h[b] = pltpu.async_copy(ra[b], y_hbm.at[pl.ds(t0, CH)], so[b])
    out_h[(nch - 1) % 2].wait()
    if out_h[nch % 2] is not None:
        out_h[nch % 2].wait()


def _combine_call(osort, dest, wba, wbb, sh):
    mesh = plsc.VectorSubcoreMesh(core_axis_name="c", subcore_axis_name="s")
    f = functools.partial(
        pl.kernel,
        out_type=jax.ShapeDtypeStruct((T, D), jnp.float32),
        mesh=mesh,
        scratch_types=(
            [pltpu.VMEM((CH,), jnp.int32)] * 4
            + [pltpu.VMEM((CH, 16), jnp.float32)] * 4
            + [pltpu.VMEM((CH, D), jnp.float32)] * 6
            + [pltpu.SemaphoreType.DMA] * 8
        ),
    )(_combine_body)
    return f(osort, dest, wba, wbb, sh)


def kernel(hidden_states, gate_weight, expert_bias, Wg, Wu, Wd, Sg, Su, Sd):
    x2d = hidden_states.reshape(T, D)
    logits, topk, dest2d, wba, wbb, counts8 = _route_call(
        x2d, gate_weight, expert_bias.reshape(1, E))
    dest = jnp.transpose(dest2d).reshape(P)
    counts = counts8[0]
    xbuf = _dispatch_call(x2d, dest)
    sh = _shared_call(x2d, Sg, Su, Sd)
    osort = _ffn_call(counts, xbuf, Wg, Wu, Wd)
    y = _combine_call(osort, dest, wba, wbb, sh)
    return (y.reshape(B, S, D), logits.reshape(B, S, E),
            topk.reshape(B, S, TOPK))


# speedup vs baseline: 1.1695x; 1.0678x over previous
import functools

import jax
import jax.numpy as jnp
from jax import lax
from jax.experimental import pallas as pl
from jax.experimental.pallas import tpu as pltpu
from jax.experimental.pallas import tpu_sc as plsc

E = 64
TOPK = 2
NG = 8
GSZ = E // NG
TKG = 4
D = 1024
FF = 256
CAP = 256
B = 1
S = 2048
T = B * S
P = T * TOPK

TB = 1024
NTB = T // TB

TILE = 128
TPE = CAP // TILE
NT = (E * CAP) // TILE
TRASH = E * CAP
XROWS = E * CAP + TILE

NWORK = 32
PPW = P // NWORK
TPW = T // NWORK
CH = 16


def _first_argmax(vals, iota, width):
    m = jnp.max(vals, axis=1, keepdims=True)
    eq = vals == m
    fidx = jnp.min(jnp.where(eq, iota, width), axis=1, keepdims=True)
    sel = iota == fidx
    return fidx, sel


def _route_body(x_ref, gw_ref, b_ref,
                lo_ref, tk_ref, de_ref, wa_ref, wb_ref, cnt_ref,
                run_ref, tril_ref):
    step = pl.program_id(0)

    @pl.when(step == 0)
    def _():
        run_ref[...] = jnp.zeros_like(run_ref)
        r_iota = lax.broadcasted_iota(jnp.int32, (TB, TB), 0)
        c_iota = lax.broadcasted_iota(jnp.int32, (TB, TB), 1)
        tril_ref[...] = (r_iota > c_iota).astype(jnp.float32)

    x = x_ref[...]
    logits = lax.dot_general(x, gw_ref[...], (((1,), (1,)), ((), ())),
                             preferred_element_type=jnp.float32)
    lo_ref[...] = logits
    scores = 1.0 / (1.0 + jnp.exp(-logits))
    s_rt = scores + b_ref[...]

    gcols = []
    for g in range(NG):
        blk = s_rt[:, g * GSZ:(g + 1) * GSZ]
        m1 = jnp.max(blk, axis=1, keepdims=True)
        eq = blk == m1
        ndup = jnp.sum(eq.astype(jnp.float32), axis=1, keepdims=True)
        m2 = jnp.max(jnp.where(eq, -jnp.inf, blk), axis=1, keepdims=True)
        m2 = jnp.where(ndup > 1.0, m1, m2)
        gcols.append(m1 + m2)
    gs = jnp.concatenate(gcols, axis=1)

    iota_g = lax.broadcasted_iota(jnp.int32, (TB, NG), 1)
    gmask = jnp.zeros((TB, NG), jnp.float32)
    for _ in range(TKG):
        _, sel = _first_argmax(gs, iota_g, NG)
        gmask = jnp.where(sel, 1.0, gmask)
        gs = jnp.where(sel, -jnp.inf, gs)

    cols = []
    for g in range(NG):
        keep = gmask[:, g:g + 1] > 0
        cols.append(jnp.where(keep, s_rt[:, g * GSZ:(g + 1) * GSZ], -jnp.inf))
    masked = jnp.concatenate(cols, axis=1)

    iota_e = lax.broadcasted_iota(jnp.int32, (TB, E), 1)
    idxs, scs = [], []
    for _ in range(TOPK):
        fidx, sel = _first_argmax(masked, iota_e, E)
        idxs.append(fidx)
        scs.append(jnp.sum(jnp.where(sel, scores, 0.0), axis=1, keepdims=True))
        masked = jnp.where(sel, -jnp.inf, masked)
    topk = jnp.concatenate(idxs, axis=1)
    tk_ref[...] = topk
    sc = jnp.concatenate(scs, axis=1)
    tw = sc / (jnp.sum(sc, axis=1, keepdims=True) + 1e-20)

    e0, e1 = idxs[0], idxs[1]
    oh0 = (iota_e == e0).astype(jnp.float32)
    oh1 = (iota_e == e1).astype(jnp.float32)
    ohsum = oh0 + oh1
    prev = jnp.dot(tril_ref[...], ohsum, preferred_element_type=jnp.float32)
    prev = prev + run_ref[...]
    rank0 = jnp.sum(prev * oh0, axis=1, keepdims=True).astype(jnp.int32)
    rank1 = jnp.sum(prev * oh1, axis=1, keepdims=True).astype(jnp.int32)
    run_ref[...] = run_ref[...] + jnp.sum(ohsum, axis=0, keepdims=True)

    v0 = rank0 < CAP
    v1 = rank1 < CAP
    dest0 = jnp.where(v0, e0 * CAP + rank0, TRASH)
    dest1 = jnp.where(v1, e1 * CAP + rank1, TRASH)
    de_ref[...] = jnp.concatenate([dest0, dest1], axis=1)
    w0 = tw[:, 0:1] * v0.astype(jnp.float32)
    w1 = tw[:, 1:2] * v1.astype(jnp.float32)
    wa_ref[...] = jnp.broadcast_to(w0, (TB, 16))
    wb_ref[...] = jnp.broadcast_to(w1, (TB, 16))
    cnt_ref[...] = jnp.broadcast_to(run_ref[...], cnt_ref.shape).astype(jnp.int32)


def _route_call(x2d, gw, bias):
    return pl.pallas_call(
        _route_body,
        grid=(NTB,),
        in_specs=[
            pl.BlockSpec((TB, D), lambda i: (i, 0)),
            pl.BlockSpec((E, D), lambda i: (0, 0)),
            pl.BlockSpec((1, E), lambda i: (0, 0)),
        ],
        out_specs=[
            pl.BlockSpec((TB, E), lambda i: (i, 0)),
            pl.BlockSpec((TB, TOPK), lambda i: (i, 0)),
            pl.BlockSpec((TB, TOPK), lambda i: (i, 0)),
            pl.BlockSpec((TB, 16), lambda i: (i, 0)),
            pl.BlockSpec((TB, 16), lambda i: (i, 0)),
            pl.BlockSpec((8, E), lambda i: (0, 0)),
        ],
        out_shape=[
            jax.ShapeDtypeStruct((T, E), jnp.float32),
            jax.ShapeDtypeStruct((T, TOPK), jnp.int32),
            jax.ShapeDtypeStruct((T, TOPK), jnp.int32),
            jax.ShapeDtypeStruct((T, 16), jnp.float32),
            jax.ShapeDtypeStruct((T, 16), jnp.float32),
            jax.ShapeDtypeStruct((8, E), jnp.int32),
        ],
        scratch_shapes=[pltpu.VMEM((1, E), jnp.float32),
                        pltpu.VMEM((TB, TB), jnp.float32)],
    )(x2d, gw, bias)


def _shared_body(x_ref, sg_ref, su_ref, sd_ref, sh_ref):
    x = x_ref[...]
    g_sh = jnp.dot(x, sg_ref[...], preferred_element_type=jnp.float32)
    u_sh = jnp.dot(x, su_ref[...], preferred_element_type=jnp.float32)
    h_sh = (g_sh * (1.0 / (1.0 + jnp.exp(-g_sh)))) * u_sh
    sh_ref[...] = jnp.dot(h_sh, sd_ref[...], preferred_element_type=jnp.float32)


def _shared_call(x2d, Sg, Su, Sd):
    return pl.pallas_call(
        _shared_body,
        grid=(NTB,),
        in_specs=[
            pl.BlockSpec((TB, D), lambda i: (i, 0)),
            pl.BlockSpec((D, FF), lambda i: (0, 0)),
            pl.BlockSpec((D, FF), lambda i: (0, 0)),
            pl.BlockSpec((FF, D), lambda i: (0, 0)),
        ],
        out_specs=pl.BlockSpec((TB, D), lambda i: (i, 0)),
        out_shape=jax.ShapeDtypeStruct((T, D), jnp.float32),
    )(x2d, Sg, Su, Sd)


def _dispatch_body(x_hbm, dest_hbm, xbuf_hbm,
                   dvm0, dvm1, rows0, rows1, sg0, sg1, ss0, ss1):
    wid = lax.axis_index("s") * 2 + lax.axis_index("c")
    lane = lax.iota(jnp.int32, 16)
    nch = PPW // CH
    dvms = [dvm0, dvm1]
    rows = [rows0, rows1]
    gsem = [sg0, sg1]
    ssem = [ss0, ss1]

    def issue_gather(c):
        base = wid * PPW + c * CH
        tok = (base + lane) & (T - 1)
        pltpu.sync_copy(dest_hbm.at[pl.ds(base, CH)], dvms[c % 2])
        return pltpu.async_copy(x_hbm.at[tok], rows[c % 2], gsem[c % 2])

    gh = [issue_gather(0), None]
    sh_ = [None, None]
    for c in range(nch):
        b = c % 2
        nb = (c + 1) % 2
        if c + 1 < nch:
            if sh_[nb] is not None:
                sh_[nb].wait()
            gh[nb] = issue_gather(c + 1)
        gh[b].wait()
        sh_[b] = pltpu.async_copy(rows[b], xbuf_hbm.at[dvms[b][...]], ssem[b])
    sh_[(nch - 1) % 2].wait()
    if sh_[nch % 2] is not None:
        sh_[nch % 2].wait()


def _dispatch_call(x2d, dest):
    mesh = plsc.VectorSubcoreMesh(core_axis_name="c", subcore_axis_name="s")
    f = functools.partial(
        pl.kernel,
        out_type=jax.ShapeDtypeStruct((XROWS, D), jnp.float32),
        mesh=mesh,
        scratch_types=[
            pltpu.VMEM((CH,), jnp.int32),
            pltpu.VMEM((CH,), jnp.int32),
            pltpu.VMEM((CH, D), jnp.float32),
            pltpu.VMEM((CH, D), jnp.float32),
            pltpu.SemaphoreType.DMA,
            pltpu.SemaphoreType.DMA,
            pltpu.SemaphoreType.DMA,
            pltpu.SemaphoreType.DMA,
        ],
    )(_dispatch_body)
    return f(x2d, dest)


def _ffn_body(cnt_ref, x_ref, wg_ref, wu_ref, wd_ref, o_ref):
    g = pl.program_id(0)
    e = jnp.minimum(g // TPE, E - 1)
    t = g % TPE
    is_trash = g >= NT
    active = jnp.logical_and(g < NT, cnt_ref[e] > t * TILE)

    @pl.when(is_trash)
    def _():
        o_ref[...] = jnp.zeros_like(o_ref)

    @pl.when(active)
    def _():
        x = x_ref[...]
        gg = jnp.dot(x, wg_ref[0], preferred_element_type=jnp.float32)
        uu = jnp.dot(x, wu_ref[0], preferred_element_type=jnp.float32)
        h = (gg * (1.0 / (1.0 + jnp.exp(-gg)))) * uu
        o_ref[...] = jnp.dot(h, wd_ref[0], preferred_element_type=jnp.float32)


def _ffn_xmap(g, cnt):
    e = jnp.minimum(g // TPE, E - 1)
    t = g % TPE
    last_act = jnp.maximum((cnt[e] + TILE - 1) // TILE - 1, 0)
    b = jnp.where(g >= NT, NT, TPE * e + jnp.minimum(t, last_act))
    return (b, 0)


def _ffn_call(counts, xbuf, Wg, Wu, Wd):
    grid_spec = pltpu.PrefetchScalarGridSpec(
        num_scalar_prefetch=1,
        grid=(NT + 1,),
        in_specs=[
            pl.BlockSpec((TILE, D), _ffn_xmap),
            pl.BlockSpec((1, D, FF), lambda g, c: (jnp.minimum(g // TPE, E - 1), 0, 0)),
            pl.BlockSpec((1, D, FF), lambda g, c: (jnp.minimum(g // TPE, E - 1), 0, 0)),
            pl.BlockSpec((1, FF, D), lambda g, c: (jnp.minimum(g // TPE, E - 1), 0, 0)),
        ],
        out_specs=pl.BlockSpec((TILE, D), _ffn_xmap),
    )
    return pl.pallas_call(
        _ffn_body,
        grid_spec=grid_spec,
        out_shape=jax.ShapeDtypeStruct((XROWS, D), jnp.float32),
    )(counts, xbuf, Wg, Wu, Wd)


def _combine_body(osort_hbm, dest_hbm, wa_hbm, wb_hbm, sh_hbm, y_hbm,
                  dva0, dva1, dvb0, dvb1, wva0, wva1, wvb0, wvb1,
                  ra0, ra1, rb0, rb1, rs0, rs1,
                  sa0, sa1, sb0, sb1, ss0, ss1, so0, so1):
    wid = lax.axis_index("s") * 2 + lax.axis_index("c")
    nch = TPW // CH
    dva = [dva0, dva1]
    dvb = [dvb0, dvb1]
    wva = [wva0, wva1]
    wvb = [wvb0, wvb1]
    ra = [ra0, ra1]
    rb = [rb0, rb1]
    rs = [rs0, rs1]
    sa = [sa0, sa1]
    sb = [sb0, sb1]
    ss = [ss0, ss1]
    so = [so0, so1]

    def issue(c):
        t0 = wid * TPW + c * CH
        b = c % 2
        pltpu.sync_copy(dest_hbm.at[pl.ds(t0, CH)], dva[b])
        pltpu.sync_copy(dest_hbm.at[pl.ds(T + t0, CH)], dvb[b])
        pltpu.sync_copy(wa_hbm.at[pl.ds(t0, CH)], wva[b])
        pltpu.sync_copy(wb_hbm.at[pl.ds(t0, CH)], wvb[b])
        return (pltpu.async_copy(osort_hbm.at[dva[b][...]], ra[b], sa[b]),
                pltpu.async_copy(osort_hbm.at[dvb[b][...]], rb[b], sb[b]),
                pltpu.async_copy(sh_hbm.at[pl.ds(t0, CH)], rs[b], ss[b]))

    hnd = [issue(0), None]
    out_h = [None, None]
    for c in range(nch):
        b = c % 2
        nb = (c + 1) % 2
        if c + 1 < nch:
            if out_h[nb] is not None:
                out_h[nb].wait()
            hnd[nb] = issue(c + 1)
        for h in hnd[b]:
            h.wait()
        for r in range(CH):
            wa = wva[b][r]
            wb = wvb[b][r]

            def body(j, _):
                for u in range(8):
                    sl = pl.ds(j * 128 + u * 16, 16)
                    ra[b][r, sl] = (wa * ra[b][r, sl] + wb * rb[b][r, sl]
                                    + rs[b][r, sl])
                return 0
            lax.fori_loop(0, D // 128, body, 0)
        t0 = wid * TPW + c * CH
        out_h[b] = pltpu.async_copy(ra[b], y_hbm.at[pl.ds(t0, CH)], so[b])
    out_h[(nch - 1) % 2].wait()
    if out_h[nch % 2] is not None:
        out_h[nch % 2].wait()


def _combine_call(osort, dest, wba, wbb, sh):
    mesh = plsc.VectorSubcoreMesh(core_axis_name="c", subcore_axis_name="s")
    f = functools.partial(
        pl.kernel,
        out_type=jax.ShapeDtypeStruct((T, D), jnp.float32),
        mesh=mesh,
        scratch_types=(
            [pltpu.VMEM((CH,), jnp.int32)] * 4
            + [pltpu.VMEM((CH, 16), jnp.float32)] * 4
            + [pltpu.VMEM((CH, D), jnp.float32)] * 6
            + [pltpu.SemaphoreType.DMA] * 8
        ),
    )(_combine_body)
    return f(osort, dest, wba, wbb, sh)


def kernel(hidden_states, gate_weight, expert_bias, Wg, Wu, Wd, Sg, Su, Sd):
    x2d = hidden_states.reshape(T, D)
    logits, topk, dest2d, wba, wbb, counts8 = _route_call(
        x2d, gate_weight, expert_bias.reshape(1, E))
    dest = jnp.transpose(dest2d).reshape(P)
    counts = counts8[0]
    xbuf = _dispatch_call(x2d, dest)
    sh = _shared_call(x2d, Sg, Su, Sd)
    osort = _ffn_call(counts, xbuf, Wg, Wu, Wd)
    y = _combine_call(osort, dest, wba, wbb, sh)
    return (y.reshape(B, S, D), logits.reshape(B, S, E),
            topk.reshape(B, S, TOPK))
